# Initial kernel scaffold; baseline (speedup 1.0000x reference)
import jax
import jax.numpy as jnp
from jax.experimental import pallas as pl


def kernel(x, edge_index, W1, b1, W2, b2):
    raise NotImplementedError("write your pallas kernel here")


# speedup vs baseline: 3.2393x; 3.2393x over previous
import functools

import jax
import jax.numpy as jnp
from jax import lax
from jax.experimental import pallas as pl
from jax.experimental.pallas import tpu as pltpu
from jax.experimental.pallas import tpu_sc as plsc

N, E, F, H, C = 10000, 320000, 128, 128, 64
NP = 10240
CHUNK = 128
EP = 327680
NC, NS = 2, 16
CHUNKS_TOTAL = EP // CHUNK
CHUNKS_PER_CORE = CHUNKS_TOTAL // NC
CHUNKS_PER_TILE = CHUNKS_PER_CORE // NS
ROWS_PER_TILE = NP // NS


def _sc_scatter(table, src2d, dst2d):
    mesh = plsc.VectorSubcoreMesh(core_axis_name="c", subcore_axis_name="s")

    @functools.partial(
        pl.kernel,
        out_type=jax.ShapeDtypeStruct((NC, NP, F), jnp.float32),
        mesh=mesh,
        scratch_types=[
            pltpu.VMEM_SHARED((NP, F), jnp.float32),
            pltpu.VMEM((CHUNKS_PER_TILE, CHUNK), jnp.int32),
            pltpu.VMEM((CHUNKS_PER_TILE, CHUNK), jnp.int32),
            pltpu.VMEM((CHUNK, F), jnp.float32),
        ],
    )
    def k(table_hbm, src_hbm, dst_hbm, out_hbm, agg_sh, src_v, dst_v, rows_v):
        c = lax.axis_index("c")
        s = lax.axis_index("s")

        @pl.loop(0, CHUNK)
        def _(i):
            for g in range(F // 16):
                rows_v[i, pl.ds(g * 16, 16)] = jnp.zeros((16,), jnp.float32)

        row0 = s * ROWS_PER_TILE
        for t in range(ROWS_PER_TILE // CHUNK):
            pltpu.sync_copy(rows_v, agg_sh.at[pl.ds(row0 + t * CHUNK, CHUNK)])
        plsc.subcore_barrier()

        base = c * CHUNKS_PER_CORE + s * CHUNKS_PER_TILE
        pltpu.sync_copy(src_hbm.at[pl.ds(base, CHUNKS_PER_TILE)], src_v)
        pltpu.sync_copy(dst_hbm.at[pl.ds(base, CHUNKS_PER_TILE)], dst_v)

        @pl.loop(0, CHUNKS_PER_TILE)
        def _(j):
            pltpu.sync_copy(table_hbm.at[src_v.at[j]], rows_v)
            pltpu.sync_copy(rows_v, agg_sh.at[dst_v.at[j]], add=True)

        plsc.subcore_barrier()
        for t in range(ROWS_PER_TILE // CHUNK):
            sl = pl.ds(row0 + t * CHUNK, CHUNK)
            pltpu.sync_copy(agg_sh.at[sl], out_hbm.at[c, sl])

    return k(table, src2d, dst2d)


_BM = 1024


def _tc_relu(x):
    def body(x_ref, o_ref):
        o_ref[...] = jnp.maximum(x_ref[...], 0.0)

    return pl.pallas_call(
        body,
        grid=(NP // _BM,),
        in_specs=[pl.BlockSpec((_BM, F), lambda i: (i, 0))],
        out_specs=pl.BlockSpec((_BM, F), lambda i: (i, 0)),
        out_shape=jax.ShapeDtypeStruct((NP, F), jnp.float32),
    )(x)


def _tc_update(x, parts, W, b, final):
    K, M = W.shape

    def body(x_ref, p_ref, w_ref, b_ref, o_ref):
        acc = x_ref[...] + p_ref[0] + p_ref[1]
        z = jax.lax.dot_general(
            acc, w_ref[...], (((1,), (0,)), ((), ())),
            precision=lax.Precision.HIGHEST,
            preferred_element_type=jnp.float32,
        ) + b_ref[...]
        if final:
            m = jnp.max(z, axis=1, keepdims=True)
            e = jnp.exp(z - m)
            o_ref[...] = (z - m) - jnp.log(jnp.sum(e, axis=1, keepdims=True))
        else:
            o_ref[...] = jnp.maximum(z, 0.0)

    return pl.pallas_call(
        body,
        grid=(NP // _BM,),
        in_specs=[
            pl.BlockSpec((_BM, K), lambda i: (i, 0)),
            pl.BlockSpec((NC, _BM, K), lambda i: (0, i, 0)),
            pl.BlockSpec((K, M), lambda i: (0, 0)),
            pl.BlockSpec((1, M), lambda i: (0, 0)),
        ],
        out_specs=pl.BlockSpec((_BM, M), lambda i: (i, 0)),
        out_shape=jax.ShapeDtypeStruct((NP, M), jnp.float32),
    )(x, parts, W, b)


def kernel(x, edge_index, W1, b1, W2, b2):
    x_p = jnp.pad(x, ((0, NP - N), (0, 0)))
    pad = jnp.full((EP - E,), N, jnp.int32)
    src2d = jnp.concatenate([edge_index[0], pad]).reshape(CHUNKS_TOTAL, CHUNK)
    dst2d = jnp.concatenate([edge_index[1], pad]).reshape(CHUNKS_TOTAL, CHUNK)

    relu_x = _tc_relu(x_p)
    parts1 = _sc_scatter(relu_x, src2d, dst2d)
    h = _tc_update(x_p, parts1, W1, b1.reshape(1, H), final=False)
    parts2 = _sc_scatter(h, src2d, dst2d)
    out = _tc_update(h, parts2, W2, b2.reshape(1, C), final=True)
    return out[:N]


# speedup vs baseline: 3.2875x; 1.0149x over previous
import functools

import jax
import jax.numpy as jnp
from jax import lax
from jax.experimental import pallas as pl
from jax.experimental.pallas import tpu as pltpu
from jax.experimental.pallas import tpu_sc as plsc

N, E, F, H, C = 10000, 320000, 128, 128, 64
NP = 10240
CHUNK = 64
EP = 327680
NC, NS = 2, 16
CHUNKS_TOTAL = EP // CHUNK
CHUNKS_PER_CORE = CHUNKS_TOTAL // NC
CHUNKS_PER_TILE = CHUNKS_PER_CORE // NS
ROWS_PER_TILE = NP // NS


def _sc_scatter(table, src2d, dst2d):
    mesh = plsc.VectorSubcoreMesh(core_axis_name="c", subcore_axis_name="s")

    @functools.partial(
        pl.kernel,
        out_type=jax.ShapeDtypeStruct((NC, NP, F), jnp.float32),
        mesh=mesh,
        scratch_types=[
            pltpu.VMEM_SHARED((NP, F), jnp.float32),
            pltpu.VMEM((CHUNKS_PER_TILE // 2, CHUNK), jnp.int32),
            pltpu.VMEM((CHUNKS_PER_TILE // 2, CHUNK), jnp.int32),
            pltpu.VMEM((2, CHUNK, F), jnp.float32),
        ] + [pltpu.SemaphoreType.DMA] * 5,
    )
    def k(table_hbm, src_hbm, dst_hbm, out_hbm,
          agg_sh, src_v, dst_v, rows_v, g0, g1, s0, s1, bsem):
        gsem = (g0, g1)
        ssem = (s0, s1)
        c = lax.axis_index("c")
        s = lax.axis_index("s")
        row0 = s * ROWS_PER_TILE
        nseg = ROWS_PER_TILE // CHUNK

        HALF = CHUNKS_PER_TILE // 2
        base = c * CHUNKS_PER_CORE + s * CHUNKS_PER_TILE

        def _load_idx(h):
            pltpu.sync_copy(src_hbm.at[pl.ds(base + h * HALF, HALF)], src_v)
            pltpu.sync_copy(dst_hbm.at[pl.ds(base + h * HALF, HALF)], dst_v)

        _load_idx(0)

        @pl.loop(0, CHUNK)
        def _(i):
            for g in range(F // 16):
                rows_v[0, i, pl.ds(g * 16, 16)] = jnp.zeros((16,), jnp.float32)
        zcp = [
            pltpu.async_copy(rows_v.at[0],
                             agg_sh.at[pl.ds(row0 + t * CHUNK, CHUNK)], bsem)
            for t in range(nseg)
        ]
        for cp in zcp:
            cp.wait()

        def _wait_gather(j, b):
            pltpu.make_async_copy(table_hbm.at[src_v.at[j]],
                                  rows_v.at[b], gsem[b]).wait()

        def _start_scatter(j, b):
            pltpu.async_copy(rows_v.at[b], agg_sh.at[dst_v.at[j]],
                             ssem[b], add=True)

        def _wait_scatter(j, b):
            pltpu.make_async_copy(rows_v.at[b], agg_sh.at[dst_v.at[j]],
                                  ssem[b]).wait()

        def _start_gather(j, b):
            pltpu.async_copy(table_hbm.at[src_v.at[j]], rows_v.at[b],
                             gsem[b])

        _start_gather(0, 0)
        plsc.subcore_barrier()

        for h in range(2):
            if h == 1:
                _start_gather(0, 0)
            _wait_gather(0, 0)
            _start_scatter(0, 0)
            _start_gather(1, 1)

            @pl.loop(1, HALF - 1, step=2)
            def _(j0):
                for u in range(2):
                    j = j0 + u
                    b = (1 + u) % 2
                    _wait_gather(j, b)
                    _start_scatter(j, b)
                    _wait_scatter(j - 1, 1 - b)
                    _start_gather(j + 1, 1 - b)

            jl = HALF - 1
            _wait_gather(jl, 1)
            _start_scatter(jl, 1)
            _wait_scatter(jl - 1, 0)
            _wait_scatter(jl, 1)
            if h == 0:
                _load_idx(1)

        plsc.subcore_barrier()

        plsc.subcore_barrier()

        wcp = []
        for t in range(nseg):
            sl = pl.ds(row0 + t * CHUNK, CHUNK)
            wcp.append(pltpu.async_copy(agg_sh.at[sl], out_hbm.at[c, sl], bsem))
        for cp in wcp:
            cp.wait()

    return k(table, src2d, dst2d)


_BM = 1024


def _tc_relu(x):
    def body(x_ref, o_ref):
        o_ref[...] = jnp.maximum(x_ref[...], 0.0)

    return pl.pallas_call(
        body,
        grid=(NP // _BM,),
        in_specs=[pl.BlockSpec((_BM, F), lambda i: (i, 0))],
        out_specs=pl.BlockSpec((_BM, F), lambda i: (i, 0)),
        out_shape=jax.ShapeDtypeStruct((NP, F), jnp.float32),
    )(x)


def _tc_update(x, parts, W, b, final):
    K, M = W.shape

    def body(x_ref, p_ref, w_ref, b_ref, o_ref):
        acc = x_ref[...] + p_ref[0] + p_ref[1]
        z = jax.lax.dot_general(
            acc, w_ref[...], (((1,), (0,)), ((), ())),
            precision=lax.Precision.HIGHEST,
            preferred_element_type=jnp.float32,
        ) + b_ref[...]
        if final:
            m = jnp.max(z, axis=1, keepdims=True)
            e = jnp.exp(z - m)
            o_ref[...] = (z - m) - jnp.log(jnp.sum(e, axis=1, keepdims=True))
        else:
            o_ref[...] = jnp.maximum(z, 0.0)

    return pl.pallas_call(
        body,
        grid=(NP // _BM,),
        in_specs=[
            pl.BlockSpec((_BM, K), lambda i: (i, 0)),
            pl.BlockSpec((NC, _BM, K), lambda i: (0, i, 0)),
            pl.BlockSpec((K, M), lambda i: (0, 0)),
            pl.BlockSpec((1, M), lambda i: (0, 0)),
        ],
        out_specs=pl.BlockSpec((_BM, M), lambda i: (i, 0)),
        out_shape=jax.ShapeDtypeStruct((NP, M), jnp.float32),
    )(x, parts, W, b)


def kernel(x, edge_index, W1, b1, W2, b2):
    x_p = jnp.pad(x, ((0, NP - N), (0, 0)))
    pad = jnp.full((EP - E,), N, jnp.int32)
    src2d = jnp.concatenate([edge_index[0], pad]).reshape(CHUNKS_TOTAL, CHUNK)
    dst2d = jnp.concatenate([edge_index[1], pad]).reshape(CHUNKS_TOTAL, CHUNK)

    relu_x = _tc_relu(x_p)
    parts1 = _sc_scatter(relu_x, src2d, dst2d)
    h = _tc_update(x_p, parts1, W1, b1.reshape(1, H), final=False)
    parts2 = _sc_scatter(h, src2d, dst2d)
    out = _tc_update(h, parts2, W2, b2.reshape(1, C), final=True)
    return out[:N]


# speedup vs baseline: 3.7510x; 1.1410x over previous
import functools

import jax
import jax.numpy as jnp
from jax import lax
from jax.experimental import pallas as pl
from jax.experimental.pallas import tpu as pltpu
from jax.experimental.pallas import tpu_sc as plsc

N, E, F, H, C = 10000, 320000, 128, 128, 64
NP = 10240
CHUNK = 64
EP = 327680
NC, NS = 2, 16
CHUNKS_TOTAL = EP // CHUNK
CHUNKS_PER_CORE = CHUNKS_TOTAL // NC
CHUNKS_PER_TILE = CHUNKS_PER_CORE // NS
ROWS_PER_TILE = NP // NS
SECT = 32
SECT0, SECT1 = 7, 3
NC0_CHUNKS = NS * SECT0 * SECT


def _sc_scatter(table, src2d, dst2d):
    mesh = plsc.VectorSubcoreMesh(core_axis_name="c", subcore_axis_name="s")

    @functools.partial(
        pl.kernel,
        out_type=jax.ShapeDtypeStruct((NC, NP, F), jnp.float32),
        mesh=mesh,
        scratch_types=[
            pltpu.VMEM_SHARED((NP, F), jnp.float32),
            pltpu.VMEM((2, SECT, CHUNK), jnp.int32),
            pltpu.VMEM((2, SECT, CHUNK), jnp.int32),
            pltpu.VMEM((2, CHUNK, F), jnp.float32),
        ] + [pltpu.SemaphoreType.DMA] * 7,
    )
    def k(table_hbm, src_hbm, dst_hbm, out_hbm,
          agg_sh, src_v, dst_v, rows_v, g0, g1, s0, s1, i0, i1, bsem):
        gsem = (g0, g1)
        ssem = (s0, s1)
        isem = (i0, i1)
        c = lax.axis_index("c")
        s = lax.axis_index("s")
        row0 = s * ROWS_PER_TILE
        nseg = ROWS_PER_TILE // CHUNK

        def _idx_copies(sect_chunk0, t):
            return (
                pltpu.make_async_copy(src_hbm.at[pl.ds(sect_chunk0, SECT)],
                                      src_v.at[t], isem[t]),
                pltpu.make_async_copy(dst_hbm.at[pl.ds(sect_chunk0, SECT)],
                                      dst_v.at[t], isem[t]),
            )

        def _wait_gather(t, l, b):
            pltpu.make_async_copy(table_hbm.at[src_v.at[t, l]],
                                  rows_v.at[b], gsem[b]).wait()

        def _start_scatter(t, l, b):
            pltpu.async_copy(rows_v.at[b], agg_sh.at[dst_v.at[t, l]],
                             ssem[b], add=True)

        def _wait_scatter(t, l, b):
            pltpu.make_async_copy(rows_v.at[b], agg_sh.at[dst_v.at[t, l]],
                                  ssem[b]).wait()

        def _start_gather(t, l, b):
            pltpu.async_copy(table_hbm.at[src_v.at[t, l]], rows_v.at[b],
                             gsem[b])

        def _run(nsect, base):
            for sect in range(nsect):
                t = sect % 2
                if sect > 0:
                    for cp in _idx_copies(base + sect * SECT, t):
                        cp.wait()
                if sect + 1 < nsect:
                    for cp in _idx_copies(base + (sect + 1) * SECT, 1 - t):
                        cp.start()

                _start_gather(t, 0, 0)
                _wait_gather(t, 0, 0)
                _start_scatter(t, 0, 0)
                _start_gather(t, 1, 1)

                @pl.loop(1, SECT - 1, step=2)
                def _(j0):
                    for u in range(2):
                        j = j0 + u
                        b = (1 + u) % 2
                        _wait_gather(t, j, b)
                        _start_scatter(t, j, b)
                        _wait_scatter(t, j - 1, 1 - b)
                        _start_gather(t, j + 1, 1 - b)

                jl = SECT - 1
                _wait_gather(t, jl, 1)
                _start_scatter(t, jl, 1)
                _wait_scatter(t, jl - 1, 0)
                _wait_scatter(t, jl, 1)

        base0 = s * SECT0 * SECT
        base1 = NC0_CHUNKS + s * SECT1 * SECT
        base_c = jnp.where(c == 0, base0, base1)
        for cp in _idx_copies(base_c, 0):
            cp.start()

        @pl.loop(0, CHUNK)
        def _(i):
            for g in range(F // 16):
                rows_v[0, i, pl.ds(g * 16, 16)] = jnp.zeros((16,), jnp.float32)
        zcp = [
            pltpu.async_copy(rows_v.at[0],
                             agg_sh.at[pl.ds(row0 + t * CHUNK, CHUNK)], bsem)
            for t in range(nseg)
        ]
        for cp in zcp:
            cp.wait()
        for cp in _idx_copies(base_c, 0):
            cp.wait()
        plsc.subcore_barrier()

        pl.when(c == 0)(lambda: _run(SECT0, base0))
        pl.when(c != 0)(lambda: _run(SECT1, base1))
        plsc.subcore_barrier()

        wcp = []
        for t in range(nseg):
            sl = pl.ds(row0 + t * CHUNK, CHUNK)
            wcp.append(pltpu.async_copy(agg_sh.at[sl], out_hbm.at[c, sl], bsem))
        for cp in wcp:
            cp.wait()

    return k(table, src2d, dst2d)


_BM = 1024


def _tc_relu(x):
    def body(x_ref, o_ref):
        o_ref[...] = jnp.maximum(x_ref[...], 0.0)

    return pl.pallas_call(
        body,
        grid=(NP // _BM,),
        in_specs=[pl.BlockSpec((_BM, F), lambda i: (i, 0))],
        out_specs=pl.BlockSpec((_BM, F), lambda i: (i, 0)),
        out_shape=jax.ShapeDtypeStruct((NP, F), jnp.float32),
    )(x)


def _tc_update(x, parts, W, b, final):
    K, M = W.shape

    def body(x_ref, p_ref, w_ref, b_ref, o_ref):
        acc = x_ref[...] + p_ref[0] + p_ref[1]
        z = jax.lax.dot_general(
            acc, w_ref[...], (((1,), (0,)), ((), ())),
            precision=lax.Precision.HIGHEST,
            preferred_element_type=jnp.float32,
        ) + b_ref[...]
        if final:
            m = jnp.max(z, axis=1, keepdims=True)
            e = jnp.exp(z - m)
            o_ref[...] = (z - m) - jnp.log(jnp.sum(e, axis=1, keepdims=True))
        else:
            o_ref[...] = jnp.maximum(z, 0.0)

    return pl.pallas_call(
        body,
        grid=(NP // _BM,),
        in_specs=[
            pl.BlockSpec((_BM, K), lambda i: (i, 0)),
            pl.BlockSpec((NC, _BM, K), lambda i: (0, i, 0)),
            pl.BlockSpec((K, M), lambda i: (0, 0)),
            pl.BlockSpec((1, M), lambda i: (0, 0)),
        ],
        out_specs=pl.BlockSpec((_BM, M), lambda i: (i, 0)),
        out_shape=jax.ShapeDtypeStruct((NP, M), jnp.float32),
    )(x, parts, W, b)


def kernel(x, edge_index, W1, b1, W2, b2):
    x_p = jnp.pad(x, ((0, NP - N), (0, 0)))
    pad = jnp.full((EP - E,), N, jnp.int32)
    src2d = jnp.concatenate([edge_index[0], pad]).reshape(CHUNKS_TOTAL, CHUNK)
    dst2d = jnp.concatenate([edge_index[1], pad]).reshape(CHUNKS_TOTAL, CHUNK)

    relu_x = _tc_relu(x_p)
    parts1 = _sc_scatter(relu_x, src2d, dst2d)
    h = _tc_update(x_p, parts1, W1, b1.reshape(1, H), final=False)
    parts2 = _sc_scatter(h, src2d, dst2d)
    out = _tc_update(h, parts2, W2, b2.reshape(1, C), final=True)
    return out[:N]


# speedup vs baseline: 4.0215x; 1.0721x over previous
import functools

import jax
import jax.numpy as jnp
from jax import lax
from jax.experimental import pallas as pl
from jax.experimental.pallas import tpu as pltpu
from jax.experimental.pallas import tpu_sc as plsc

N, E, F, H, C = 10000, 320000, 128, 128, 64
NP = 10240
CHUNK = 64
EP = 327680
NC, NS = 2, 16
CHUNKS_TOTAL = EP // CHUNK
CHUNKS_PER_CORE = CHUNKS_TOTAL // NC
CHUNKS_PER_TILE = CHUNKS_PER_CORE // NS
ROWS_PER_TILE = NP // NS
SECT = 32
SECT0, SECT1 = 8, 2
NC0_CHUNKS = NS * SECT0 * SECT


def _sc_scatter(table, src2d, dst2d):
    mesh = plsc.VectorSubcoreMesh(core_axis_name="c", subcore_axis_name="s")

    @functools.partial(
        pl.kernel,
        out_type=jax.ShapeDtypeStruct((NC, NP, F), jnp.float32),
        mesh=mesh,
        scratch_types=[
            pltpu.VMEM_SHARED((NP, F), jnp.float32),
            pltpu.VMEM((2, SECT, CHUNK), jnp.int32),
            pltpu.VMEM((2, SECT, CHUNK), jnp.int32),
            pltpu.VMEM((2, CHUNK, F), jnp.float32),
        ] + [pltpu.SemaphoreType.DMA] * 7,
    )
    def k(table_hbm, src_hbm, dst_hbm, out_hbm,
          agg_sh, src_v, dst_v, rows_v, g0, g1, s0, s1, i0, i1, bsem):
        gsem = (g0, g1)
        ssem = (s0, s1)
        isem = (i0, i1)
        c = lax.axis_index("c")
        s = lax.axis_index("s")
        row0 = s * ROWS_PER_TILE
        nseg = ROWS_PER_TILE // CHUNK

        def _idx_copies(sect_chunk0, t):
            return (
                pltpu.make_async_copy(src_hbm.at[pl.ds(sect_chunk0, SECT)],
                                      src_v.at[t], isem[t]),
                pltpu.make_async_copy(dst_hbm.at[pl.ds(sect_chunk0, SECT)],
                                      dst_v.at[t], isem[t]),
            )

        def _wait_gather(t, l, b):
            pltpu.make_async_copy(table_hbm.at[src_v.at[t, l]],
                                  rows_v.at[b], gsem[b]).wait()

        def _start_scatter(t, l, b):
            pltpu.async_copy(rows_v.at[b], agg_sh.at[dst_v.at[t, l]],
                             ssem[b], add=True)

        def _wait_scatter(t, l, b):
            pltpu.make_async_copy(rows_v.at[b], agg_sh.at[dst_v.at[t, l]],
                                  ssem[b]).wait()

        def _start_gather(t, l, b):
            pltpu.async_copy(table_hbm.at[src_v.at[t, l]], rows_v.at[b],
                             gsem[b])

        def _run(nsect, base):
            for sect in range(nsect):
                t = sect % 2
                if sect > 0:
                    for cp in _idx_copies(base + sect * SECT, t):
                        cp.wait()
                if sect + 1 < nsect:
                    for cp in _idx_copies(base + (sect + 1) * SECT, 1 - t):
                        cp.start()

                _start_gather(t, 0, 0)
                _wait_gather(t, 0, 0)
                _start_scatter(t, 0, 0)
                _start_gather(t, 1, 1)

                @pl.loop(1, SECT - 1, step=2)
                def _(j0):
                    for u in range(2):
                        j = j0 + u
                        b = (1 + u) % 2
                        _wait_gather(t, j, b)
                        _start_scatter(t, j, b)
                        _wait_scatter(t, j - 1, 1 - b)
                        _start_gather(t, j + 1, 1 - b)

                jl = SECT - 1
                _wait_gather(t, jl, 1)
                _start_scatter(t, jl, 1)
                _wait_scatter(t, jl - 1, 0)
                _wait_scatter(t, jl, 1)

        base0 = s * SECT0 * SECT
        base1 = NC0_CHUNKS + s * SECT1 * SECT
        base_c = jnp.where(c == 0, base0, base1)
        for cp in _idx_copies(base_c, 0):
            cp.start()

        @pl.loop(0, CHUNK)
        def _(i):
            for g in range(F // 16):
                rows_v[0, i, pl.ds(g * 16, 16)] = jnp.zeros((16,), jnp.float32)
        zcp = [
            pltpu.async_copy(rows_v.at[0],
                             agg_sh.at[pl.ds(row0 + t * CHUNK, CHUNK)], bsem)
            for t in range(nseg)
        ]
        for cp in zcp:
            cp.wait()
        for cp in _idx_copies(base_c, 0):
            cp.wait()
        plsc.subcore_barrier()

        pl.when(c == 0)(lambda: _run(SECT0, base0))
        pl.when(c != 0)(lambda: _run(SECT1, base1))
        plsc.subcore_barrier()

        wcp = []
        for t in range(nseg):
            sl = pl.ds(row0 + t * CHUNK, CHUNK)
            wcp.append(pltpu.async_copy(agg_sh.at[sl], out_hbm.at[c, sl], bsem))
        for cp in wcp:
            cp.wait()

    return k(table, src2d, dst2d)


_BM = 1024


def _tc_relu(x):
    def body(x_ref, o_ref):
        o_ref[...] = jnp.maximum(x_ref[...], 0.0)

    return pl.pallas_call(
        body,
        grid=(NP // _BM,),
        in_specs=[pl.BlockSpec((_BM, F), lambda i: (i, 0))],
        out_specs=pl.BlockSpec((_BM, F), lambda i: (i, 0)),
        out_shape=jax.ShapeDtypeStruct((NP, F), jnp.float32),
    )(x)


def _tc_update(x, parts, W, b, final):
    K, M = W.shape

    def body(x_ref, p_ref, w_ref, b_ref, o_ref):
        acc = x_ref[...] + p_ref[0] + p_ref[1]
        z = jax.lax.dot_general(
            acc, w_ref[...], (((1,), (0,)), ((), ())),
            precision=lax.Precision.HIGHEST,
            preferred_element_type=jnp.float32,
        ) + b_ref[...]
        if final:
            m = jnp.max(z, axis=1, keepdims=True)
            e = jnp.exp(z - m)
            o_ref[...] = (z - m) - jnp.log(jnp.sum(e, axis=1, keepdims=True))
        else:
            o_ref[...] = jnp.maximum(z, 0.0)

    return pl.pallas_call(
        body,
        grid=(NP // _BM,),
        in_specs=[
            pl.BlockSpec((_BM, K), lambda i: (i, 0)),
            pl.BlockSpec((NC, _BM, K), lambda i: (0, i, 0)),
            pl.BlockSpec((K, M), lambda i: (0, 0)),
            pl.BlockSpec((1, M), lambda i: (0, 0)),
        ],
        out_specs=pl.BlockSpec((_BM, M), lambda i: (i, 0)),
        out_shape=jax.ShapeDtypeStruct((NP, M), jnp.float32),
    )(x, parts, W, b)


def kernel(x, edge_index, W1, b1, W2, b2):
    x_p = jnp.pad(x, ((0, NP - N), (0, 0)))
    pad = jnp.full((EP - E,), N, jnp.int32)
    src2d = jnp.concatenate([edge_index[0], pad]).reshape(CHUNKS_TOTAL, CHUNK)
    dst2d = jnp.concatenate([edge_index[1], pad]).reshape(CHUNKS_TOTAL, CHUNK)

    relu_x = _tc_relu(x_p)
    parts1 = _sc_scatter(relu_x, src2d, dst2d)
    h = _tc_update(x_p, parts1, W1, b1.reshape(1, H), final=False)
    parts2 = _sc_scatter(h, src2d, dst2d)
    out = _tc_update(h, parts2, W2, b2.reshape(1, C), final=True)
    return out[:N]
